# baseline (speedup 1.0000x reference)
import functools

import jax
import jax.numpy as jnp
from jax.experimental import pallas as pl
from jax.experimental.pallas import tpu as pltpu


def _make_kernel(B, S, D, Rb):
    NS = S // Rb

    def _ln_kernel(tab_ref, g_ref, b_ref, out_ref, x_ref, y_ref, sem_ref):
        s = pl.program_id(0)
        b = pl.program_id(1)
        slot = jax.lax.rem(s, 2)

        @pl.when((s == 0) & (b == 0))
        def _start_first():
            pltpu.make_async_copy(
                tab_ref.at[pl.ds(0, Rb), :], x_ref.at[0], sem_ref.at[0]
            ).start()

        @pl.when(b == 0)
        def _compute():
            pltpu.make_async_copy(
                tab_ref.at[pl.ds(s * Rb, Rb), :], x_ref.at[slot], sem_ref.at[slot]
            ).wait()
            x = x_ref[slot]
            mean = jnp.mean(x, axis=-1, keepdims=True)
            xc = x - mean
            var = jnp.mean(xc * xc, axis=-1, keepdims=True)
            y_ref[...] = xc * jax.lax.rsqrt(var + 1e-5) * g_ref[...] + b_ref[...]

            @pl.when(s + 1 < NS)
            def _prefetch_next():
                pltpu.make_async_copy(
                    tab_ref.at[pl.ds((s + 1) * Rb, Rb), :],
                    x_ref.at[1 - slot],
                    sem_ref.at[1 - slot],
                ).start()

        out_ref[...] = y_ref[...][None]

    return _ln_kernel, NS


@functools.partial(jax.jit, static_argnames=("interpret",))
def _run(inputs, table, gamma, beta, interpret=False):
    B, S = inputs.shape
    D = table.shape[1]
    Rb = 512 if S % 512 == 0 else S
    body, NS = _make_kernel(B, S, D, Rb)
    g2 = gamma.reshape(1, D)
    b2 = beta.reshape(1, D)
    return pl.pallas_call(
        body,
        grid=(NS, B),
        in_specs=[
            pl.BlockSpec(memory_space=pltpu.MemorySpace.HBM),
            pl.BlockSpec((1, D), lambda s, b: (0, 0)),
            pl.BlockSpec((1, D), lambda s, b: (0, 0)),
        ],
        out_specs=pl.BlockSpec((1, Rb, D), lambda s, b: (b, s, 0)),
        out_shape=jax.ShapeDtypeStruct((B, S, D), table.dtype),
        scratch_shapes=[
            pltpu.VMEM((2, Rb, D), table.dtype),
            pltpu.VMEM((Rb, D), table.dtype),
            pltpu.SemaphoreType.DMA((2,)),
        ],
        compiler_params=pltpu.CompilerParams(
            dimension_semantics=("arbitrary", "arbitrary"),
        ),
        interpret=interpret,
    )(table, g2, b2)


def kernel(inputs, table, gamma, beta):
    return _run(inputs, table, gamma, beta)


# speedup vs baseline: 1.0312x; 1.0312x over previous
import functools

import jax
import jax.numpy as jnp
from jax.experimental import pallas as pl
from jax.experimental.pallas import tpu as pltpu


def _make_kernel(B, S, D, Rb):
    NS = S // Rb

    def _ln_kernel(tab_ref, g_ref, b_ref, out_ref, x_ref, y_ref, sem_ref):
        s = pl.program_id(0)
        b = pl.program_id(1)

        @pl.when((s == 0) & (b == 0))
        def _start_fetches():
            for s2 in range(NS):
                pltpu.make_async_copy(
                    tab_ref.at[pl.ds(s2 * Rb, Rb), :], x_ref.at[s2], sem_ref.at[s2]
                ).start()

        for s2 in range(NS):

            @pl.when((s == s2) & (b == 0))
            def _compute(s2=s2):
                pltpu.make_async_copy(
                    tab_ref.at[pl.ds(s2 * Rb, Rb), :], x_ref.at[s2], sem_ref.at[s2]
                ).wait()
                x = x_ref[s2]
                mean = jnp.mean(x, axis=-1, keepdims=True)
                xc = x - mean
                var = jnp.mean(xc * xc, axis=-1, keepdims=True)
                y_ref[...] = xc * jax.lax.rsqrt(var + 1e-5) * g_ref[...] + b_ref[...]

        out_ref[...] = y_ref[...][None]

    return _ln_kernel, NS


@functools.partial(jax.jit, static_argnames=("interpret",))
def _run(inputs, table, gamma, beta, interpret=False):
    B, S = inputs.shape
    D = table.shape[1]
    Rb = 512 if S % 512 == 0 else S
    body, NS = _make_kernel(B, S, D, Rb)
    g2 = gamma.reshape(1, D)
    b2 = beta.reshape(1, D)
    return pl.pallas_call(
        body,
        grid=(NS, B),
        in_specs=[
            pl.BlockSpec(memory_space=pltpu.MemorySpace.HBM),
            pl.BlockSpec((1, D), lambda s, b: (0, 0)),
            pl.BlockSpec((1, D), lambda s, b: (0, 0)),
        ],
        out_specs=pl.BlockSpec((1, Rb, D), lambda s, b: (b, s, 0)),
        out_shape=jax.ShapeDtypeStruct((B, S, D), table.dtype),
        scratch_shapes=[
            pltpu.VMEM((NS, Rb, D), table.dtype),
            pltpu.VMEM((Rb, D), table.dtype),
            pltpu.SemaphoreType.DMA((NS,)),
        ],
        compiler_params=pltpu.CompilerParams(
            dimension_semantics=("arbitrary", "arbitrary"),
        ),
        interpret=interpret,
    )(table, g2, b2)


def kernel(inputs, table, gamma, beta):
    return _run(inputs, table, gamma, beta)


# speedup vs baseline: 1.1380x; 1.1036x over previous
import functools

import jax
import jax.numpy as jnp
from jax.experimental import pallas as pl
from jax.experimental.pallas import tpu as pltpu


def _make_ln_kernel(B, S, D, Rb):
    NS = S // Rb

    def _ln_kernel(tab_ref, g_ref, b_ref, out_ref, y_ref, sem_ref):
        s = pl.program_id(0)
        x = tab_ref[...]
        mean = jnp.mean(x, axis=-1, keepdims=True)
        xc = x - mean
        var = jnp.mean(xc * xc, axis=-1, keepdims=True)
        y_ref[s] = xc * jax.lax.rsqrt(var + 1e-5) * g_ref[...] + b_ref[...]
        for bi in range(B):
            pltpu.make_async_copy(
                y_ref.at[s],
                out_ref.at[bi, pl.ds(s * Rb, Rb), :],
                sem_ref.at[s, bi],
            ).start()

        @pl.when(s == NS - 1)
        def _drain():
            for s2 in range(NS):
                for bi in range(B):
                    pltpu.make_async_copy(
                        y_ref.at[s2],
                        out_ref.at[bi, pl.ds(s2 * Rb, Rb), :],
                        sem_ref.at[s2, bi],
                    ).wait()

    return _ln_kernel, NS


@functools.partial(jax.jit, static_argnames=("interpret",))
def _run(inputs, table, gamma, beta, interpret=False):
    B, S = inputs.shape
    D = table.shape[1]
    Rb = 512 if S % 512 == 0 else S
    body, NS = _make_ln_kernel(B, S, D, Rb)
    g2 = gamma.reshape(1, D)
    b2 = beta.reshape(1, D)
    return pl.pallas_call(
        body,
        grid=(NS,),
        in_specs=[
            pl.BlockSpec((Rb, D), lambda s: (s, 0)),
            pl.BlockSpec((1, D), lambda s: (0, 0)),
            pl.BlockSpec((1, D), lambda s: (0, 0)),
        ],
        out_specs=pl.BlockSpec(memory_space=pltpu.MemorySpace.HBM),
        out_shape=jax.ShapeDtypeStruct((B, S, D), table.dtype),
        scratch_shapes=[
            pltpu.VMEM((NS, Rb, D), table.dtype),
            pltpu.SemaphoreType.DMA((NS, B)),
        ],
        compiler_params=pltpu.CompilerParams(
            dimension_semantics=("arbitrary",),
        ),
        interpret=interpret,
    )(table, g2, b2)


def kernel(inputs, table, gamma, beta):
    return _run(inputs, table, gamma, beta)
